# Initial kernel scaffold; baseline (speedup 1.0000x reference)
import jax
import jax.numpy as jnp
from jax.experimental import pallas as pl


def kernel(inputs, std_edge, W1, b1, W2, b2, W3, b3, W4, b4, Wfc, bfc, Wr, br):
    raise NotImplementedError("write your pallas kernel here")


# speedup vs baseline: 11.1797x; 11.1797x over previous
import functools

import jax
import jax.numpy as jnp
from jax import lax
from jax.experimental import pallas as pl
from jax.experimental.pallas import tpu as pltpu
from jax.experimental.pallas import tpu_sc as plsc

N = 2048
B = 4
D = 128
E = 65536

TS = 512
NT = N // TS


_NW = 32
_EPW = E // _NW
_G = 128
_NCH = _EPW // _G
_R = N // 16


def _sc_mesh():
    return plsc.VectorSubcoreMesh(core_axis_name="c", subcore_axis_name="s")


def _sc_msg(yflat, src_w, dst_w, zeros_rows):
    nb = src_w.shape[0]

    @functools.partial(
        pl.kernel,
        mesh=_sc_mesh(),
        out_type=jax.ShapeDtypeStruct((2, nb, N, D), jnp.float32),
        scratch_types=[
            pltpu.VMEM_SHARED((N, D), jnp.float32),
            pltpu.VMEM((_G, D), jnp.float32),
            pltpu.VMEM((_G,), jnp.int32),
            pltpu.VMEM((_G,), jnp.int32),
            pltpu.SemaphoreType.DMA,
        ],
    )
    def kern(y_hbm, src_hbm, dst_hbm, zero_hbm, out_hbm,
             acc, rows, srcv, dstv, sem):
        ci = lax.axis_index("c")
        si = lax.axis_index("s")
        wid = ci * 16 + si
        for b in range(nb):
            pltpu.sync_copy(zero_hbm, acc.at[pl.ds(si * _R, _R)])
            plsc.subcore_barrier()

            def chunk(c, carry):
                pltpu.sync_copy(src_hbm.at[b, wid, c], srcv)
                pltpu.sync_copy(dst_hbm.at[wid, c], dstv)
                pltpu.async_copy(y_hbm.at[srcv], rows, sem).wait()
                pltpu.sync_copy(rows, acc.at[dstv], add=True)
                return carry

            lax.fori_loop(0, _NCH, chunk, 0)
            plsc.subcore_barrier()
            pltpu.sync_copy(acc.at[pl.ds(si * _R, _R)],
                            out_hbm.at[ci, b, pl.ds(si * _R, _R)])

    return kern(yflat, src_w, dst_w, zeros_rows)


def _edge_finalize(acc, y, dinv, bias_row, *, relu):
    nb = y.shape[0]

    def body(a_ref, y_ref, d_ref, b_ref, o_ref):
        for b in range(nb):
            z = a_ref[0, b] + a_ref[1, b] + y_ref[b]
            z = d_ref[...] * z + b_ref[...]
            if relu:
                z = jnp.maximum(z, 0.0)
            o_ref[b] = z

    return pl.pallas_call(
        body,
        grid=(NT,),
        in_specs=[
            pl.BlockSpec((2, nb, TS, D), lambda i: (0, 0, i, 0)),
            pl.BlockSpec((nb, TS, D), lambda i: (0, i, 0)),
            pl.BlockSpec((TS, 1), lambda i: (i, 0)),
            pl.BlockSpec((1, D), lambda i: (0, 0)),
        ],
        out_specs=pl.BlockSpec((nb, TS, D), lambda i: (0, i, 0)),
        out_shape=jax.ShapeDtypeStruct((nb, N, D), jnp.float32),
    )(acc, y, dinv, bias_row)


def _linear_rowscale(x, wt, scale):
    M = x.shape[0]
    TM = 1024

    def body(x_ref, w_ref, s_ref, o_ref):
        o_ref[...] = s_ref[...] * jnp.dot(
            x_ref[...], w_ref[...], preferred_element_type=jnp.float32)

    return pl.pallas_call(
        body,
        grid=(M // TM,),
        in_specs=[
            pl.BlockSpec((TM, D), lambda i: (i, 0)),
            pl.BlockSpec((D, D), lambda i: (0, 0)),
            pl.BlockSpec((TM, 1), lambda i: (i, 0)),
        ],
        out_specs=pl.BlockSpec((TM, D), lambda i: (i, 0)),
        out_shape=jax.ShapeDtypeStruct((M, D), jnp.float32),
    )(x, wt, scale)


def _adj_stats(x, xt, wfc, bfc_col):
    def body(x_ref, xt_ref, w_ref, bf_ref, acc_ref):
        j = pl.program_id(1)
        i = pl.program_id(2)
        xtb = xt_ref[0]
        pt = jnp.dot(w_ref[...], xtb, preferred_element_type=jnp.float32)
        pt = jnp.maximum(pt + bf_ref[...], 0.0)
        xj = x_ref[0]
        rn = lax.rsqrt(jnp.sum(xj * xj, axis=1, keepdims=True))
        cn = lax.rsqrt(jnp.sum(xtb * xtb, axis=0, keepdims=True))
        sim = rn * jnp.dot(xj, xtb, preferred_element_type=jnp.float32) * cn
        s2 = jnp.sum(pt * pt)
        mx = jnp.max(pt)
        c1 = jnp.sum(pt * sim)
        c2 = jnp.sum(sim * sim)
        lane = lax.broadcasted_iota(jnp.int32, (1, 8, 128), 2)
        sumc = jnp.where(lane == 0, s2,
                         jnp.where(lane == 2, c1,
                                   jnp.where(lane == 3, c2, 0.0)))
        mxc = jnp.where(lane == 1, mx, 0.0)
        first = jnp.logical_and(j == 0, i == 0)
        prev = jnp.where(first, 0.0, acc_ref[...])
        acc_ref[...] = jnp.where(lane == 1, jnp.maximum(prev, mxc),
                                 prev + sumc)

    return pl.pallas_call(
        body,
        grid=(B, NT, NT),
        in_specs=[
            pl.BlockSpec((1, TS, D), lambda b, j, i: (b, j, 0)),
            pl.BlockSpec((1, D, TS), lambda b, j, i: (b, 0, i)),
            pl.BlockSpec((TS, D), lambda b, j, i: (j, 0)),
            pl.BlockSpec((TS, 1), lambda b, j, i: (j, 0)),
        ],
        out_specs=pl.BlockSpec((1, 8, 128), lambda b, j, i: (b, 0, 0)),
        out_shape=jax.ShapeDtypeStruct((B, 8, 128), jnp.float32),
    )(x, xt, wfc, bfc_col)


def _adj_tile(w_ref, xt_ref, bf_ref, th_ref, j, i):
    pt = jnp.dot(w_ref[...], xt_ref[0], preferred_element_type=jnp.float32)
    pt = jnp.maximum(pt + bf_ref[...], 0.0)
    a = (pt >= th_ref[0, 0, 0]).astype(jnp.float32)
    cg = j * TS + lax.broadcasted_iota(jnp.int32, (TS, TS), 0)
    rg = i * TS + lax.broadcasted_iota(jnp.int32, (TS, TS), 1)
    return jnp.maximum(a, (cg == rg).astype(jnp.float32))


def _mask_deg(xt, wfc, bfc_col, thresh):
    def body(xt_ref, w_ref, bf_ref, th_ref, deg_ref):
        j = pl.program_id(1)
        i = pl.program_id(2)
        a = _adj_tile(w_ref, xt_ref, bf_ref, th_ref, j, i)
        contrib = jnp.sum(a, axis=1, keepdims=True)

        @pl.when(i == 0)
        def _():
            deg_ref[...] = contrib

        @pl.when(i > 0)
        def _():
            deg_ref[...] += contrib

    return pl.pallas_call(
        body,
        grid=(B, NT, NT),
        in_specs=[
            pl.BlockSpec((1, D, TS), lambda b, j, i: (b, 0, i)),
            pl.BlockSpec((TS, D), lambda b, j, i: (j, 0)),
            pl.BlockSpec((TS, 1), lambda b, j, i: (j, 0)),
            pl.BlockSpec((1, 1, 1), lambda b, j, i: (b, 0, 0)),
        ],
        out_specs=pl.BlockSpec((TS, 1), lambda b, j, i: (b * NT + j, 0)),
        out_shape=jax.ShapeDtypeStruct((B * N, 1), jnp.float32),
    )(xt, wfc, bfc_col, thresh)


def _mask_agg(xt, wfc, bfc_col, thresh, y, dinv_col, bias_row, *, relu):
    def body(xt_ref, w_ref, bf_ref, th_ref, y_ref, d_ref, b_ref, o_ref):
        j = pl.program_id(1)
        i = pl.program_id(2)
        a = _adj_tile(w_ref, xt_ref, bf_ref, th_ref, j, i)
        contrib = jnp.dot(a, y_ref[0], preferred_element_type=jnp.float32)

        @pl.when(i == 0)
        def _():
            o_ref[0] = contrib

        @pl.when(i > 0)
        def _():
            o_ref[0] += contrib

        @pl.when(i == NT - 1)
        def _():
            z = d_ref[...] * o_ref[0] + b_ref[...]
            if relu:
                z = jnp.maximum(z, 0.0)
            o_ref[0] = z

    return pl.pallas_call(
        body,
        grid=(B, NT, NT),
        in_specs=[
            pl.BlockSpec((1, D, TS), lambda b, j, i: (b, 0, i)),
            pl.BlockSpec((TS, D), lambda b, j, i: (j, 0)),
            pl.BlockSpec((TS, 1), lambda b, j, i: (j, 0)),
            pl.BlockSpec((1, 1, 1), lambda b, j, i: (b, 0, 0)),
            pl.BlockSpec((1, TS, D), lambda b, j, i: (b, i, 0)),
            pl.BlockSpec((TS, 1), lambda b, j, i: (b * NT + j, 0)),
            pl.BlockSpec((1, D), lambda b, j, i: (0, 0)),
        ],
        out_specs=pl.BlockSpec((1, TS, D), lambda b, j, i: (b, j, 0)),
        out_shape=jax.ShapeDtypeStruct((B, N, D), jnp.float32),
    )(xt, wfc, bfc_col, thresh, y, dinv_col, bias_row)


def _final_readout(agg0, y4, b4_row, wr_pad, br_pad):
    TT = 512

    def body(a_ref, y_ref, b4_ref, wr_ref, br_ref, o_ref):
        i = pl.program_id(0)
        x4 = jnp.where(i < NT, a_ref[...], y_ref[...] + b4_ref[...])
        o_ref[...] = jnp.dot(x4, wr_ref[...],
                             preferred_element_type=jnp.float32) + br_ref[...]

    return pl.pallas_call(
        body,
        grid=(B * N // TT,),
        in_specs=[
            pl.BlockSpec((TT, D), lambda i: (i % NT, 0)),
            pl.BlockSpec((TT, D), lambda i: (i, 0)),
            pl.BlockSpec((1, D), lambda i: (0, 0)),
            pl.BlockSpec((D, 128), lambda i: (0, 0)),
            pl.BlockSpec((1, 128), lambda i: (0, 0)),
        ],
        out_specs=pl.BlockSpec((TT, 128), lambda i: (i, 0)),
        out_shape=jax.ShapeDtypeStruct((B * N, 128), jnp.float32),
    )(agg0, y4, b4_row, wr_pad, br_pad)


def _adjacency_scalars(acc3):
    acc = acc3[:, 0, :]
    s2, mx, c1, c2 = acc[:, 0], acc[:, 1], acc[:, 2], acc[:, 3]
    rs2 = jnp.sqrt(s2)
    gmax = jnp.max(mx / rs2)
    thresh = (0.5 * gmax) * rs2
    loss = jnp.mean(1.0 - 2.0 * (c1 / rs2) + c2)
    return thresh.reshape(B, 1, 1), loss


def kernel(inputs, std_edge, W1, b1, W2, b2, W3, b3, W4, b4, Wfc, bfc, Wr, br):
    x0 = inputs.astype(jnp.float32)
    src = std_edge[0]
    dst = std_edge[1]

    dst_w = dst.reshape(_NW, _NCH, _G)
    zrows0 = jnp.zeros((_R, D), jnp.float32)
    degp = _sc_msg(jnp.ones((N, D), jnp.float32),
                   src.reshape(1, _NW, _NCH, _G), dst_w, zrows0)
    deg_std = degp[0, 0, :, 0] + degp[1, 0, :, 0] + 1.0
    dinv_std = (deg_std ** -0.5).reshape(N, 1)
    dinv_std_b = jnp.tile(dinv_std, (B, 1))

    bfc_col = bfc.reshape(N, 1)
    zrows = zrows0

    y1 = _linear_rowscale(x0.reshape(B * N, D), W1.T, dinv_std_b)
    offs = (jnp.arange(B, dtype=src.dtype) * N)[:, None]
    src_wb = (src[None, :] + offs).reshape(B, _NW, _NCH, _G)
    macc1 = _sc_msg(y1, src_wb, dst_w, zrows)
    x1 = _edge_finalize(macc1, y1.reshape(B, N, D), dinv_std,
                        b1.reshape(1, D), relu=True)

    xt1 = x1.transpose(0, 2, 1)
    acc1 = _adj_stats(x1, xt1, Wfc, bfc_col)
    thresh1, loss1 = _adjacency_scalars(acc1)
    deg2 = _mask_deg(xt1, Wfc, bfc_col, thresh1)
    dinv2 = deg2 ** -0.5
    y2 = _linear_rowscale(x1.reshape(B * N, D), W2.T, dinv2)
    x2 = _mask_agg(xt1, Wfc, bfc_col, thresh1, y2.reshape(B, N, D), dinv2,
                   b2.reshape(1, D), relu=True)

    xt2 = x2.transpose(0, 2, 1)
    acc2 = _adj_stats(x2, xt2, Wfc, bfc_col)
    thresh2, loss2 = _adjacency_scalars(acc2)
    deg3 = _mask_deg(xt2, Wfc, bfc_col, thresh2)
    dinv3 = deg3 ** -0.5
    y3 = _linear_rowscale(x2.reshape(B * N, D), W3.T, dinv3)
    x3 = _mask_agg(xt2, Wfc, bfc_col, thresh2, y3.reshape(B, N, D), dinv3,
                   b3.reshape(1, D), relu=True)

    scale4 = jnp.concatenate(
        [dinv_std, jnp.ones(((B - 1) * N, 1), jnp.float32)], axis=0)
    y4 = _linear_rowscale(x3.reshape(B * N, D), W4.T, scale4)
    macc4 = _sc_msg(y4[:N], src.reshape(1, _NW, _NCH, _G), dst_w, zrows)
    agg0 = _edge_finalize(macc4, y4[:N].reshape(1, N, D), dinv_std,
                          b4.reshape(1, D), relu=False)[0]

    wr_pad = jnp.zeros((D, 128), jnp.float32).at[:, :2].set(Wr.T)
    br_pad = jnp.zeros((1, 128), jnp.float32).at[0, :2].set(br)
    ro = _final_readout(agg0, y4, b4.reshape(1, D), wr_pad, br_pad)
    out = ro[:, :2].reshape(B, N, 2)
    return out, loss1 + loss2


# speedup vs baseline: 13.7149x; 1.2268x over previous
import functools

import jax
import jax.numpy as jnp
from jax import lax
from jax.experimental import pallas as pl
from jax.experimental.pallas import tpu as pltpu
from jax.experimental.pallas import tpu_sc as plsc

N = 2048
B = 4
D = 128
E = 65536

TS = 512
NT = N // TS


_NW = 32
_EPW = E // _NW
_G = 128
_NCH = _EPW // _G
_R = N // 16


def _sc_mesh():
    return plsc.VectorSubcoreMesh(core_axis_name="c", subcore_axis_name="s")


def _sc_seg_sum(yflat, idx_w, zeros_rows, ones_rows, *, gather):
    nb = idx_w.shape[0]

    @functools.partial(
        pl.kernel,
        mesh=_sc_mesh(),
        out_type=jax.ShapeDtypeStruct((2, nb, N, D), jnp.float32),
        scratch_types=[
            pltpu.VMEM_SHARED((N, D), jnp.float32),
            pltpu.VMEM((_G, D), jnp.float32),
            pltpu.VMEM((_G, D), jnp.float32),
            pltpu.VMEM((2, _G), jnp.int32),
            pltpu.VMEM((2, _G), jnp.int32),
            pltpu.SemaphoreType.DMA,
            pltpu.SemaphoreType.DMA,
        ],
    )
    def kern(y_hbm, idx_hbm, zero_hbm, ones_hbm, out_hbm,
             acc, rows0, rows1, idx0, idx1, sem0, sem1):
        ci = lax.axis_index("c")
        si = lax.axis_index("s")
        wid = ci * 16 + si
        if not gather:
            pltpu.sync_copy(ones_hbm, rows0)
        for b in range(nb):
            pltpu.sync_copy(zero_hbm, acc.at[pl.ds(si * _R, _R)])
            plsc.subcore_barrier()
            if gather:
                pltpu.sync_copy(idx_hbm.at[b, wid, 0], idx0)
                pltpu.async_copy(y_hbm.at[idx0.at[0]], rows0, sem0)

                def pair(k, carry):
                    c = 2 * k
                    pltpu.sync_copy(idx_hbm.at[b, wid, c + 1], idx1)
                    pltpu.async_copy(y_hbm.at[idx1.at[0]], rows1, sem1)
                    pltpu.make_async_copy(
                        y_hbm.at[idx0.at[0]], rows0, sem0).wait()
                    pltpu.sync_copy(rows0, acc.at[idx0.at[1]], add=True)

                    @pl.when(k < _NCH // 2 - 1)
                    def _():
                        pltpu.sync_copy(idx_hbm.at[b, wid, c + 2], idx0)
                        pltpu.async_copy(y_hbm.at[idx0.at[0]], rows0, sem0)

                    pltpu.make_async_copy(
                        y_hbm.at[idx1.at[0]], rows1, sem1).wait()
                    pltpu.sync_copy(rows1, acc.at[idx1.at[1]], add=True)
                    return carry

                lax.fori_loop(0, _NCH // 2, pair, 0)
            else:
                def chunk(c, carry):
                    pltpu.sync_copy(idx_hbm.at[b, wid, c], idx0)
                    pltpu.sync_copy(rows0, acc.at[idx0.at[1]], add=True)
                    return carry

                lax.fori_loop(0, _NCH, chunk, 0)
            plsc.subcore_barrier()
            pltpu.sync_copy(acc.at[pl.ds(si * _R, _R)],
                            out_hbm.at[ci, b, pl.ds(si * _R, _R)])

    return kern(yflat, idx_w, zeros_rows, ones_rows)


def _edge_finalize(acc, y, dinv, bias_row, *, relu):
    nb = y.shape[0]

    def body(a_ref, y_ref, d_ref, b_ref, o_ref):
        for b in range(nb):
            z = a_ref[0, b] + a_ref[1, b] + y_ref[b]
            z = d_ref[...] * z + b_ref[...]
            if relu:
                z = jnp.maximum(z, 0.0)
            o_ref[b] = z

    return pl.pallas_call(
        body,
        grid=(NT,),
        in_specs=[
            pl.BlockSpec((2, nb, TS, D), lambda i: (0, 0, i, 0)),
            pl.BlockSpec((nb, TS, D), lambda i: (0, i, 0)),
            pl.BlockSpec((TS, 1), lambda i: (i, 0)),
            pl.BlockSpec((1, D), lambda i: (0, 0)),
        ],
        out_specs=pl.BlockSpec((nb, TS, D), lambda i: (0, i, 0)),
        out_shape=jax.ShapeDtypeStruct((nb, N, D), jnp.float32),
    )(acc, y, dinv, bias_row)


def _linear_rowscale(x, wt, scale):
    M = x.shape[0]
    TM = 1024

    def body(x_ref, w_ref, s_ref, o_ref):
        o_ref[...] = s_ref[...] * jnp.dot(
            x_ref[...], w_ref[...], preferred_element_type=jnp.float32)

    return pl.pallas_call(
        body,
        grid=(M // TM,),
        in_specs=[
            pl.BlockSpec((TM, D), lambda i: (i, 0)),
            pl.BlockSpec((D, D), lambda i: (0, 0)),
            pl.BlockSpec((TM, 1), lambda i: (i, 0)),
        ],
        out_specs=pl.BlockSpec((TM, D), lambda i: (i, 0)),
        out_shape=jax.ShapeDtypeStruct((M, D), jnp.float32),
    )(x, wt, scale)


def _adj_stats(x, xt, wfc, bfc_col):
    def body(x_ref, xt_ref, w_ref, bf_ref, acc_ref):
        j = pl.program_id(1)
        i = pl.program_id(2)
        xtb = xt_ref[0]
        pt = jnp.dot(w_ref[...], xtb, preferred_element_type=jnp.float32)
        pt = jnp.maximum(pt + bf_ref[...], 0.0)
        xj = x_ref[0]
        rn = lax.rsqrt(jnp.sum(xj * xj, axis=1, keepdims=True))
        cn = lax.rsqrt(jnp.sum(xtb * xtb, axis=0, keepdims=True))
        sim = rn * jnp.dot(xj, xtb, preferred_element_type=jnp.float32) * cn
        s2 = jnp.sum(pt * pt)
        mx = jnp.max(pt)
        c1 = jnp.sum(pt * sim)
        c2 = jnp.sum(sim * sim)
        lane = lax.broadcasted_iota(jnp.int32, (1, 8, 128), 2)
        sumc = jnp.where(lane == 0, s2,
                         jnp.where(lane == 2, c1,
                                   jnp.where(lane == 3, c2, 0.0)))
        mxc = jnp.where(lane == 1, mx, 0.0)
        first = jnp.logical_and(j == 0, i == 0)
        prev = jnp.where(first, 0.0, acc_ref[...])
        acc_ref[...] = jnp.where(lane == 1, jnp.maximum(prev, mxc),
                                 prev + sumc)

    return pl.pallas_call(
        body,
        grid=(B, NT, NT),
        in_specs=[
            pl.BlockSpec((1, TS, D), lambda b, j, i: (b, j, 0)),
            pl.BlockSpec((1, D, TS), lambda b, j, i: (b, 0, i)),
            pl.BlockSpec((TS, D), lambda b, j, i: (j, 0)),
            pl.BlockSpec((TS, 1), lambda b, j, i: (j, 0)),
        ],
        out_specs=pl.BlockSpec((1, 8, 128), lambda b, j, i: (b, 0, 0)),
        out_shape=jax.ShapeDtypeStruct((B, 8, 128), jnp.float32),
    )(x, xt, wfc, bfc_col)


def _adj_tile(w_ref, xt_ref, bf_ref, th_ref, j, i):
    pt = jnp.dot(w_ref[...], xt_ref[0], preferred_element_type=jnp.float32)
    pt = jnp.maximum(pt + bf_ref[...], 0.0)
    a = (pt >= th_ref[0, 0, 0]).astype(jnp.float32)
    cg = j * TS + lax.broadcasted_iota(jnp.int32, (TS, TS), 0)
    rg = i * TS + lax.broadcasted_iota(jnp.int32, (TS, TS), 1)
    return jnp.maximum(a, (cg == rg).astype(jnp.float32))


def _mask_deg(xt, wfc, bfc_col, thresh):
    def body(xt_ref, w_ref, bf_ref, th_ref, deg_ref):
        j = pl.program_id(1)
        i = pl.program_id(2)
        a = _adj_tile(w_ref, xt_ref, bf_ref, th_ref, j, i)
        contrib = jnp.sum(a, axis=1, keepdims=True)

        @pl.when(i == 0)
        def _():
            deg_ref[...] = contrib

        @pl.when(i > 0)
        def _():
            deg_ref[...] += contrib

    return pl.pallas_call(
        body,
        grid=(B, NT, NT),
        in_specs=[
            pl.BlockSpec((1, D, TS), lambda b, j, i: (b, 0, i)),
            pl.BlockSpec((TS, D), lambda b, j, i: (j, 0)),
            pl.BlockSpec((TS, 1), lambda b, j, i: (j, 0)),
            pl.BlockSpec((1, 1, 1), lambda b, j, i: (b, 0, 0)),
        ],
        out_specs=pl.BlockSpec((TS, 1), lambda b, j, i: (b * NT + j, 0)),
        out_shape=jax.ShapeDtypeStruct((B * N, 1), jnp.float32),
    )(xt, wfc, bfc_col, thresh)


def _mask_agg(xt, wfc, bfc_col, thresh, y, dinv_col, bias_row, *, relu):
    def body(xt_ref, w_ref, bf_ref, th_ref, y_ref, d_ref, b_ref, o_ref):
        j = pl.program_id(1)
        i = pl.program_id(2)
        a = _adj_tile(w_ref, xt_ref, bf_ref, th_ref, j, i)
        contrib = jnp.dot(a, y_ref[0], preferred_element_type=jnp.float32)

        @pl.when(i == 0)
        def _():
            o_ref[0] = contrib

        @pl.when(i > 0)
        def _():
            o_ref[0] += contrib

        @pl.when(i == NT - 1)
        def _():
            z = d_ref[...] * o_ref[0] + b_ref[...]
            if relu:
                z = jnp.maximum(z, 0.0)
            o_ref[0] = z

    return pl.pallas_call(
        body,
        grid=(B, NT, NT),
        in_specs=[
            pl.BlockSpec((1, D, TS), lambda b, j, i: (b, 0, i)),
            pl.BlockSpec((TS, D), lambda b, j, i: (j, 0)),
            pl.BlockSpec((TS, 1), lambda b, j, i: (j, 0)),
            pl.BlockSpec((1, 1, 1), lambda b, j, i: (b, 0, 0)),
            pl.BlockSpec((1, TS, D), lambda b, j, i: (b, i, 0)),
            pl.BlockSpec((TS, 1), lambda b, j, i: (b * NT + j, 0)),
            pl.BlockSpec((1, D), lambda b, j, i: (0, 0)),
        ],
        out_specs=pl.BlockSpec((1, TS, D), lambda b, j, i: (b, j, 0)),
        out_shape=jax.ShapeDtypeStruct((B, N, D), jnp.float32),
    )(xt, wfc, bfc_col, thresh, y, dinv_col, bias_row)


def _final_readout(agg0, y4, b4_row, wr_pad, br_pad):
    TT = 512

    def body(a_ref, y_ref, b4_ref, wr_ref, br_ref, o_ref):
        i = pl.program_id(0)
        x4 = jnp.where(i < NT, a_ref[...], y_ref[...] + b4_ref[...])
        o_ref[...] = jnp.dot(x4, wr_ref[...],
                             preferred_element_type=jnp.float32) + br_ref[...]

    return pl.pallas_call(
        body,
        grid=(B * N // TT,),
        in_specs=[
            pl.BlockSpec((TT, D), lambda i: (i % NT, 0)),
            pl.BlockSpec((TT, D), lambda i: (i, 0)),
            pl.BlockSpec((1, D), lambda i: (0, 0)),
            pl.BlockSpec((D, 128), lambda i: (0, 0)),
            pl.BlockSpec((1, 128), lambda i: (0, 0)),
        ],
        out_specs=pl.BlockSpec((TT, 128), lambda i: (i, 0)),
        out_shape=jax.ShapeDtypeStruct((B * N, 128), jnp.float32),
    )(agg0, y4, b4_row, wr_pad, br_pad)


def _adjacency_scalars(acc3):
    acc = acc3[:, 0, :]
    s2, mx, c1, c2 = acc[:, 0], acc[:, 1], acc[:, 2], acc[:, 3]
    rs2 = jnp.sqrt(s2)
    gmax = jnp.max(mx / rs2)
    thresh = (0.5 * gmax) * rs2
    loss = jnp.mean(1.0 - 2.0 * (c1 / rs2) + c2)
    return thresh.reshape(B, 1, 1), loss


def kernel(inputs, std_edge, W1, b1, W2, b2, W3, b3, W4, b4, Wfc, bfc, Wr, br):
    x0 = inputs.astype(jnp.float32)
    src = std_edge[0]
    dst = std_edge[1]

    dst_w = dst.reshape(_NW, _NCH, _G)
    zrows0 = jnp.zeros((_R, D), jnp.float32)
    ones_rows = jnp.ones((_G, D), jnp.float32)
    idx_deg = jnp.stack([dst_w, dst_w], axis=-2)[None]
    degp = _sc_seg_sum(ones_rows, idx_deg, zrows0, ones_rows, gather=False)
    deg_std = degp[0, 0, :, 0] + degp[1, 0, :, 0] + 1.0
    dinv_std = (deg_std ** -0.5).reshape(N, 1)
    dinv_std_b = jnp.tile(dinv_std, (B, 1))

    bfc_col = bfc.reshape(N, 1)
    zrows = zrows0

    y1 = _linear_rowscale(x0.reshape(B * N, D), W1.T, dinv_std_b)
    offs = (jnp.arange(B, dtype=src.dtype) * N)[:, None]
    src_wb = (src[None, :] + offs).reshape(B, _NW, _NCH, _G)
    idx1_w = jnp.stack(
        [src_wb, jnp.broadcast_to(dst_w[None], src_wb.shape)], axis=-2)
    macc1 = _sc_seg_sum(y1, idx1_w, zrows, ones_rows, gather=True)
    x1 = _edge_finalize(macc1, y1.reshape(B, N, D), dinv_std,
                        b1.reshape(1, D), relu=True)

    xt1 = x1.transpose(0, 2, 1)
    acc1 = _adj_stats(x1, xt1, Wfc, bfc_col)
    thresh1, loss1 = _adjacency_scalars(acc1)
    deg2 = _mask_deg(xt1, Wfc, bfc_col, thresh1)
    dinv2 = deg2 ** -0.5
    y2 = _linear_rowscale(x1.reshape(B * N, D), W2.T, dinv2)
    x2 = _mask_agg(xt1, Wfc, bfc_col, thresh1, y2.reshape(B, N, D), dinv2,
                   b2.reshape(1, D), relu=True)

    xt2 = x2.transpose(0, 2, 1)
    acc2 = _adj_stats(x2, xt2, Wfc, bfc_col)
    thresh2, loss2 = _adjacency_scalars(acc2)
    deg3 = _mask_deg(xt2, Wfc, bfc_col, thresh2)
    dinv3 = deg3 ** -0.5
    y3 = _linear_rowscale(x2.reshape(B * N, D), W3.T, dinv3)
    x3 = _mask_agg(xt2, Wfc, bfc_col, thresh2, y3.reshape(B, N, D), dinv3,
                   b3.reshape(1, D), relu=True)

    scale4 = jnp.concatenate(
        [dinv_std, jnp.ones(((B - 1) * N, 1), jnp.float32)], axis=0)
    y4 = _linear_rowscale(x3.reshape(B * N, D), W4.T, scale4)
    idx4_w = jnp.stack([src.reshape(_NW, _NCH, _G), dst_w], axis=-2)[None]
    macc4 = _sc_seg_sum(y4[:N], idx4_w, zrows, ones_rows, gather=True)
    agg0 = _edge_finalize(macc4, y4[:N].reshape(1, N, D), dinv_std,
                          b4.reshape(1, D), relu=False)[0]

    wr_pad = jnp.zeros((D, 128), jnp.float32).at[:, :2].set(Wr.T)
    br_pad = jnp.zeros((1, 128), jnp.float32).at[0, :2].set(br)
    ro = _final_readout(agg0, y4, b4.reshape(1, D), wr_pad, br_pad)
    out = ro[:, :2].reshape(B, N, 2)
    return out, loss1 + loss2


# speedup vs baseline: 17.8604x; 1.3023x over previous
import functools

import jax
import jax.numpy as jnp
from jax import lax
from jax.experimental import pallas as pl
from jax.experimental.pallas import tpu as pltpu
from jax.experimental.pallas import tpu_sc as plsc

N = 2048
B = 4
D = 128
E = 65536

TS = 1024
NT = N // TS


_NW = 32
_EPW = E // _NW
_G = 128
_NCH = _EPW // _G
_R = N // 16


def _sc_mesh():
    return plsc.VectorSubcoreMesh(core_axis_name="c", subcore_axis_name="s")


def _sc_seg_sum(yflat, idx_w, zeros_rows, ones_rows, *, gather):
    nb = idx_w.shape[0]

    @functools.partial(
        pl.kernel,
        mesh=_sc_mesh(),
        out_type=jax.ShapeDtypeStruct((2, nb, N, D), jnp.float32),
        scratch_types=[
            pltpu.VMEM_SHARED((N, D), jnp.float32),
            pltpu.VMEM((_G, D), jnp.float32),
            pltpu.VMEM((_G, D), jnp.float32),
            pltpu.VMEM((2, _G), jnp.int32),
            pltpu.VMEM((2, _G), jnp.int32),
            pltpu.SemaphoreType.DMA,
            pltpu.SemaphoreType.DMA,
        ],
    )
    def kern(y_hbm, idx_hbm, zero_hbm, ones_hbm, out_hbm,
             acc, rows0, rows1, idx0, idx1, sem0, sem1):
        ci = lax.axis_index("c")
        si = lax.axis_index("s")
        wid = ci * 16 + si
        if not gather:
            pltpu.sync_copy(ones_hbm, rows0)
        for b in range(nb):
            pltpu.sync_copy(zero_hbm, acc.at[pl.ds(si * _R, _R)])
            plsc.subcore_barrier()
            if gather:
                pltpu.sync_copy(idx_hbm.at[b, wid, 0], idx0)
                pltpu.async_copy(y_hbm.at[idx0.at[0]], rows0, sem0)

                def pair(k, carry):
                    c = 2 * k
                    pltpu.sync_copy(idx_hbm.at[b, wid, c + 1], idx1)
                    pltpu.async_copy(y_hbm.at[idx1.at[0]], rows1, sem1)
                    pltpu.make_async_copy(
                        y_hbm.at[idx0.at[0]], rows0, sem0).wait()
                    pltpu.sync_copy(rows0, acc.at[idx0.at[1]], add=True)

                    @pl.when(k < _NCH // 2 - 1)
                    def _():
                        pltpu.sync_copy(idx_hbm.at[b, wid, c + 2], idx0)
                        pltpu.async_copy(y_hbm.at[idx0.at[0]], rows0, sem0)

                    pltpu.make_async_copy(
                        y_hbm.at[idx1.at[0]], rows1, sem1).wait()
                    pltpu.sync_copy(rows1, acc.at[idx1.at[1]], add=True)
                    return carry

                lax.fori_loop(0, _NCH // 2, pair, 0)
            else:
                def chunk(c, carry):
                    pltpu.sync_copy(idx_hbm.at[b, wid, c], idx0)
                    pltpu.sync_copy(rows0, acc.at[idx0.at[1]], add=True)
                    return carry

                lax.fori_loop(0, _NCH, chunk, 0)
            plsc.subcore_barrier()
            pltpu.sync_copy(acc.at[pl.ds(si * _R, _R)],
                            out_hbm.at[ci, b, pl.ds(si * _R, _R)])

    return kern(yflat, idx_w, zeros_rows, ones_rows)


def _edge_finalize(acc, y, dinv, bias_row, *, relu):
    nb = y.shape[0]

    def body(a_ref, y_ref, d_ref, b_ref, o_ref):
        for b in range(nb):
            z = a_ref[0, b] + a_ref[1, b] + y_ref[b]
            z = d_ref[...] * z + b_ref[...]
            if relu:
                z = jnp.maximum(z, 0.0)
            o_ref[b] = z

    return pl.pallas_call(
        body,
        grid=(NT,),
        in_specs=[
            pl.BlockSpec((2, nb, TS, D), lambda i: (0, 0, i, 0)),
            pl.BlockSpec((nb, TS, D), lambda i: (0, i, 0)),
            pl.BlockSpec((TS, 1), lambda i: (i, 0)),
            pl.BlockSpec((1, D), lambda i: (0, 0)),
        ],
        out_specs=pl.BlockSpec((nb, TS, D), lambda i: (0, i, 0)),
        out_shape=jax.ShapeDtypeStruct((nb, N, D), jnp.float32),
    )(acc, y, dinv, bias_row)


def _linear_rowscale(x, wt, scale):
    M = x.shape[0]
    TM = 1024

    def body(x_ref, w_ref, s_ref, o_ref):
        o_ref[...] = s_ref[...] * jnp.dot(
            x_ref[...], w_ref[...], preferred_element_type=jnp.float32)

    return pl.pallas_call(
        body,
        grid=(M // TM,),
        in_specs=[
            pl.BlockSpec((TM, D), lambda i: (i, 0)),
            pl.BlockSpec((D, D), lambda i: (0, 0)),
            pl.BlockSpec((TM, 1), lambda i: (i, 0)),
        ],
        out_specs=pl.BlockSpec((TM, D), lambda i: (i, 0)),
        out_shape=jax.ShapeDtypeStruct((M, D), jnp.float32),
    )(x, wt, scale)


def _adj_stats(x, xt, wfc, bfc_col):
    def body(x_ref, xt_ref, w_ref, bf_ref, acc_ref):
        j = pl.program_id(1)
        i = pl.program_id(2)
        xtb = xt_ref[0]
        pt = jnp.dot(w_ref[...], xtb, preferred_element_type=jnp.float32)
        pt = jnp.maximum(pt + bf_ref[...], 0.0)
        xj = x_ref[0]
        rn = lax.rsqrt(jnp.sum(xj * xj, axis=1, keepdims=True))
        cn = lax.rsqrt(jnp.sum(xtb * xtb, axis=0, keepdims=True))
        sim = rn * jnp.dot(xj, xtb, preferred_element_type=jnp.float32) * cn
        s2 = jnp.sum(pt * pt)
        mx = jnp.max(pt)
        c1 = jnp.sum(pt * sim)
        c2 = jnp.sum(sim * sim)
        lane = lax.broadcasted_iota(jnp.int32, (1, 8, 128), 2)
        sumc = jnp.where(lane == 0, s2,
                         jnp.where(lane == 2, c1,
                                   jnp.where(lane == 3, c2, 0.0)))
        mxc = jnp.where(lane == 1, mx, 0.0)
        first = jnp.logical_and(j == 0, i == 0)
        prev = jnp.where(first, 0.0, acc_ref[...])
        acc_ref[...] = jnp.where(lane == 1, jnp.maximum(prev, mxc),
                                 prev + sumc)

    return pl.pallas_call(
        body,
        grid=(B, NT, NT),
        in_specs=[
            pl.BlockSpec((1, TS, D), lambda b, j, i: (b, j, 0)),
            pl.BlockSpec((1, D, TS), lambda b, j, i: (b, 0, i)),
            pl.BlockSpec((TS, D), lambda b, j, i: (j, 0)),
            pl.BlockSpec((TS, 1), lambda b, j, i: (j, 0)),
        ],
        out_specs=pl.BlockSpec((1, 8, 128), lambda b, j, i: (b, 0, 0)),
        out_shape=jax.ShapeDtypeStruct((B, 8, 128), jnp.float32),
    )(x, xt, wfc, bfc_col)


def _adj_tile(w_ref, xt_ref, bf_ref, th_ref, eye_ref, j, i):
    pt = jnp.dot(w_ref[...], xt_ref[0], preferred_element_type=jnp.float32)
    pt = jnp.maximum(pt + bf_ref[...], 0.0)
    a = (pt >= th_ref[0, 0, 0]).astype(jnp.float32)
    sel = jnp.where(j == i, 1.0, 0.0)
    return jnp.maximum(a, eye_ref[...] * sel)


def _mask_deg(xt, wfc, bfc_col, thresh, eye):
    def body(xt_ref, w_ref, bf_ref, th_ref, eye_ref, deg_ref):
        j = pl.program_id(1)
        i = pl.program_id(2)
        a = _adj_tile(w_ref, xt_ref, bf_ref, th_ref, eye_ref, j, i)
        contrib = jnp.sum(a, axis=1, keepdims=True)

        @pl.when(i == 0)
        def _():
            deg_ref[...] = contrib

        @pl.when(i > 0)
        def _():
            deg_ref[...] += contrib

    return pl.pallas_call(
        body,
        grid=(B, NT, NT),
        in_specs=[
            pl.BlockSpec((1, D, TS), lambda b, j, i: (b, 0, i)),
            pl.BlockSpec((TS, D), lambda b, j, i: (j, 0)),
            pl.BlockSpec((TS, 1), lambda b, j, i: (j, 0)),
            pl.BlockSpec((1, 1, 1), lambda b, j, i: (b, 0, 0)),
            pl.BlockSpec((TS, TS), lambda b, j, i: (0, 0)),
        ],
        out_specs=pl.BlockSpec((TS, 1), lambda b, j, i: (b * NT + j, 0)),
        out_shape=jax.ShapeDtypeStruct((B * N, 1), jnp.float32),
    )(xt, wfc, bfc_col, thresh, eye)


def _mask_agg(xt, wfc, bfc_col, thresh, eye, y, dinv_col, bias_row, *, relu):
    def body(xt_ref, w_ref, bf_ref, th_ref, eye_ref, y_ref, d_ref, b_ref, o_ref):
        j = pl.program_id(1)
        i = pl.program_id(2)
        a = _adj_tile(w_ref, xt_ref, bf_ref, th_ref, eye_ref, j, i)
        contrib = jnp.dot(a, y_ref[0], preferred_element_type=jnp.float32)

        @pl.when(i == 0)
        def _():
            o_ref[0] = contrib

        @pl.when(i > 0)
        def _():
            o_ref[0] += contrib

        @pl.when(i == NT - 1)
        def _():
            z = d_ref[...] * o_ref[0] + b_ref[...]
            if relu:
                z = jnp.maximum(z, 0.0)
            o_ref[0] = z

    return pl.pallas_call(
        body,
        grid=(B, NT, NT),
        in_specs=[
            pl.BlockSpec((1, D, TS), lambda b, j, i: (b, 0, i)),
            pl.BlockSpec((TS, D), lambda b, j, i: (j, 0)),
            pl.BlockSpec((TS, 1), lambda b, j, i: (j, 0)),
            pl.BlockSpec((1, 1, 1), lambda b, j, i: (b, 0, 0)),
            pl.BlockSpec((TS, TS), lambda b, j, i: (0, 0)),
            pl.BlockSpec((1, TS, D), lambda b, j, i: (b, i, 0)),
            pl.BlockSpec((TS, 1), lambda b, j, i: (b * NT + j, 0)),
            pl.BlockSpec((1, D), lambda b, j, i: (0, 0)),
        ],
        out_specs=pl.BlockSpec((1, TS, D), lambda b, j, i: (b, j, 0)),
        out_shape=jax.ShapeDtypeStruct((B, N, D), jnp.float32),
    )(xt, wfc, bfc_col, thresh, eye, y, dinv_col, bias_row)


def _final_readout(agg0, y4, b4_row, wr_pad, br_pad):
    TT = 512
    NB0 = N // TT

    def body(a_ref, y_ref, b4_ref, wr_ref, br_ref, o_ref):
        i = pl.program_id(0)
        x4 = jnp.where(i < NB0, a_ref[...], y_ref[...] + b4_ref[...])
        o_ref[...] = jnp.dot(x4, wr_ref[...],
                             preferred_element_type=jnp.float32) + br_ref[...]

    return pl.pallas_call(
        body,
        grid=(B * N // TT,),
        in_specs=[
            pl.BlockSpec((TT, D), lambda i: (i % (N // TT), 0)),
            pl.BlockSpec((TT, D), lambda i: (i, 0)),
            pl.BlockSpec((1, D), lambda i: (0, 0)),
            pl.BlockSpec((D, 128), lambda i: (0, 0)),
            pl.BlockSpec((1, 128), lambda i: (0, 0)),
        ],
        out_specs=pl.BlockSpec((TT, 128), lambda i: (i, 0)),
        out_shape=jax.ShapeDtypeStruct((B * N, 128), jnp.float32),
    )(agg0, y4, b4_row, wr_pad, br_pad)


def _adjacency_scalars(acc3):
    acc = acc3[:, 0, :]
    s2, mx, c1, c2 = acc[:, 0], acc[:, 1], acc[:, 2], acc[:, 3]
    rs2 = jnp.sqrt(s2)
    gmax = jnp.max(mx / rs2)
    thresh = (0.5 * gmax) * rs2
    loss = jnp.mean(1.0 - 2.0 * (c1 / rs2) + c2)
    return thresh.reshape(B, 1, 1), loss


def kernel(inputs, std_edge, W1, b1, W2, b2, W3, b3, W4, b4, Wfc, bfc, Wr, br):
    x0 = inputs.astype(jnp.float32)
    src = std_edge[0]
    dst = std_edge[1]

    dst_w = dst.reshape(_NW, _NCH, _G)
    zrows0 = jnp.zeros((_R, D), jnp.float32)
    ones_rows = jnp.ones((_G, D), jnp.float32)
    idx_deg = jnp.stack([dst_w, dst_w], axis=-2)[None]
    degp = _sc_seg_sum(ones_rows, idx_deg, zrows0, ones_rows, gather=False)
    deg_std = degp[0, 0, :, 0] + degp[1, 0, :, 0] + 1.0
    dinv_std = (deg_std ** -0.5).reshape(N, 1)
    dinv_std_b = jnp.tile(dinv_std, (B, 1))

    bfc_col = bfc.reshape(N, 1)
    eye_ts = jnp.eye(TS, dtype=jnp.float32)
    zrows = zrows0

    y1 = _linear_rowscale(x0.reshape(B * N, D), W1.T, dinv_std_b)
    offs = (jnp.arange(B, dtype=src.dtype) * N)[:, None]
    src_wb = (src[None, :] + offs).reshape(B, _NW, _NCH, _G)
    idx1_w = jnp.stack(
        [src_wb, jnp.broadcast_to(dst_w[None], src_wb.shape)], axis=-2)
    macc1 = _sc_seg_sum(y1, idx1_w, zrows, ones_rows, gather=True)
    x1 = _edge_finalize(macc1, y1.reshape(B, N, D), dinv_std,
                        b1.reshape(1, D), relu=True)

    xt1 = x1.transpose(0, 2, 1)
    acc1 = _adj_stats(x1, xt1, Wfc, bfc_col)
    thresh1, loss1 = _adjacency_scalars(acc1)
    deg2 = _mask_deg(xt1, Wfc, bfc_col, thresh1, eye_ts)
    dinv2 = deg2 ** -0.5
    y2 = _linear_rowscale(x1.reshape(B * N, D), W2.T, dinv2)
    x2 = _mask_agg(xt1, Wfc, bfc_col, thresh1, eye_ts, y2.reshape(B, N, D),
                   dinv2, b2.reshape(1, D), relu=True)

    xt2 = x2.transpose(0, 2, 1)
    acc2 = _adj_stats(x2, xt2, Wfc, bfc_col)
    thresh2, loss2 = _adjacency_scalars(acc2)
    deg3 = _mask_deg(xt2, Wfc, bfc_col, thresh2, eye_ts)
    dinv3 = deg3 ** -0.5
    y3 = _linear_rowscale(x2.reshape(B * N, D), W3.T, dinv3)
    x3 = _mask_agg(xt2, Wfc, bfc_col, thresh2, eye_ts, y3.reshape(B, N, D),
                   dinv3, b3.reshape(1, D), relu=True)

    scale4 = jnp.concatenate(
        [dinv_std, jnp.ones(((B - 1) * N, 1), jnp.float32)], axis=0)
    y4 = _linear_rowscale(x3.reshape(B * N, D), W4.T, scale4)
    idx4_w = jnp.stack([src.reshape(_NW, _NCH, _G), dst_w], axis=-2)[None]
    macc4 = _sc_seg_sum(y4[:N], idx4_w, zrows, ones_rows, gather=True)
    agg0 = _edge_finalize(macc4, y4[:N].reshape(1, N, D), dinv_std,
                          b4.reshape(1, D), relu=False)[0]

    wr_pad = jnp.zeros((D, 128), jnp.float32).at[:, :2].set(Wr.T)
    br_pad = jnp.zeros((1, 128), jnp.float32).at[0, :2].set(br)
    ro = _final_readout(agg0, y4, b4.reshape(1, D), wr_pad, br_pad)
    out = ro[:, :2].reshape(B, N, 2)
    return out, loss1 + loss2


# speedup vs baseline: 17.8711x; 1.0006x over previous
import functools

import jax
import jax.numpy as jnp
from jax import lax
from jax.experimental import pallas as pl
from jax.experimental.pallas import tpu as pltpu
from jax.experimental.pallas import tpu_sc as plsc

N = 2048
B = 4
D = 128
E = 65536

TS = 1024
NT = N // TS


_NW = 32
_EPW = E // _NW
_G = 128
_NCH = _EPW // _G
_R = N // 16


def _sc_mesh():
    return plsc.VectorSubcoreMesh(core_axis_name="c", subcore_axis_name="s")


def _sc_seg_sum(yflat, idx_w, zeros_rows, ones_rows, *, gather, F=D, G=_G):
    nb = idx_w.shape[0]
    nch = _EPW // G
    P = F // 128
    row_shape = (N, 128) if P == 1 else (N, P, 128)
    buf_shape = (G, 128) if P == 1 else (G, P, 128)
    out_shape = (2, nb) + row_shape

    @functools.partial(
        pl.kernel,
        mesh=_sc_mesh(),
        out_type=jax.ShapeDtypeStruct(out_shape, jnp.float32),
        scratch_types=[
            pltpu.VMEM_SHARED(row_shape, jnp.float32),
            pltpu.VMEM(buf_shape, jnp.float32),
            pltpu.VMEM(buf_shape, jnp.float32),
            pltpu.VMEM((2, G), jnp.int32),
            pltpu.VMEM((2, G), jnp.int32),
            pltpu.SemaphoreType.DMA,
            pltpu.SemaphoreType.DMA,
        ],
    )
    def kern(y_hbm, idx_hbm, zero_hbm, ones_hbm, out_hbm,
             acc, rows0, rows1, idx0, idx1, sem0, sem1):
        ci = lax.axis_index("c")
        si = lax.axis_index("s")
        wid = ci * 16 + si
        if not gather:
            pltpu.sync_copy(ones_hbm, rows0)
        for b in range(nb):
            pltpu.sync_copy(zero_hbm, acc.at[pl.ds(si * _R, _R)])
            plsc.subcore_barrier()
            if gather:
                pltpu.sync_copy(idx_hbm.at[b, wid, 0], idx0)
                pltpu.async_copy(y_hbm.at[idx0.at[0]], rows0, sem0)

                def pair(k, carry):
                    c = 2 * k
                    pltpu.sync_copy(idx_hbm.at[b, wid, c + 1], idx1)
                    pltpu.async_copy(y_hbm.at[idx1.at[0]], rows1, sem1)
                    pltpu.make_async_copy(
                        y_hbm.at[idx0.at[0]], rows0, sem0).wait()
                    pltpu.sync_copy(rows0, acc.at[idx0.at[1]], add=True)

                    @pl.when(k < nch // 2 - 1)
                    def _():
                        pltpu.sync_copy(idx_hbm.at[b, wid, c + 2], idx0)
                        pltpu.async_copy(y_hbm.at[idx0.at[0]], rows0, sem0)

                    pltpu.make_async_copy(
                        y_hbm.at[idx1.at[0]], rows1, sem1).wait()
                    pltpu.sync_copy(rows1, acc.at[idx1.at[1]], add=True)
                    return carry

                lax.fori_loop(0, nch // 2, pair, 0)
            else:
                def chunk(c, carry):
                    pltpu.sync_copy(idx_hbm.at[b, wid, c], idx0)
                    pltpu.sync_copy(rows0, acc.at[idx0.at[1]], add=True)
                    return carry

                lax.fori_loop(0, nch, chunk, 0)
            plsc.subcore_barrier()
            pltpu.sync_copy(acc.at[pl.ds(si * _R, _R)],
                            out_hbm.at[ci, b, pl.ds(si * _R, _R)])

    return kern(yflat, idx_w, zeros_rows, ones_rows)


def _edge_finalize(acc, y, dinv, bias_row, *, relu):
    nb = y.shape[0]

    def body(a_ref, y_ref, d_ref, b_ref, o_ref):
        for b in range(nb):
            z = a_ref[0, b] + a_ref[1, b] + y_ref[b]
            z = d_ref[...] * z + b_ref[...]
            if relu:
                z = jnp.maximum(z, 0.0)
            o_ref[b] = z

    return pl.pallas_call(
        body,
        grid=(NT,),
        in_specs=[
            pl.BlockSpec((2, nb, TS, D), lambda i: (0, 0, i, 0)),
            pl.BlockSpec((nb, TS, D), lambda i: (0, i, 0)),
            pl.BlockSpec((TS, 1), lambda i: (i, 0)),
            pl.BlockSpec((1, D), lambda i: (0, 0)),
        ],
        out_specs=pl.BlockSpec((nb, TS, D), lambda i: (0, i, 0)),
        out_shape=jax.ShapeDtypeStruct((nb, N, D), jnp.float32),
    )(acc, y, dinv, bias_row)


def _linear_rowscale(x, wt, scale):
    M = x.shape[0]
    TM = 1024

    def body(x_ref, w_ref, s_ref, o_ref):
        o_ref[...] = s_ref[...] * jnp.dot(
            x_ref[...], w_ref[...], preferred_element_type=jnp.float32)

    return pl.pallas_call(
        body,
        grid=(M // TM,),
        in_specs=[
            pl.BlockSpec((TM, D), lambda i: (i, 0)),
            pl.BlockSpec((D, D), lambda i: (0, 0)),
            pl.BlockSpec((TM, 1), lambda i: (i, 0)),
        ],
        out_specs=pl.BlockSpec((TM, D), lambda i: (i, 0)),
        out_shape=jax.ShapeDtypeStruct((M, D), jnp.float32),
    )(x, wt, scale)


def _adj_stats(x, xt, wfc, bfc_col):
    def body(x_ref, xt_ref, w_ref, bf_ref, acc_ref):
        j = pl.program_id(1)
        i = pl.program_id(2)
        xtb = xt_ref[0]
        pt = jnp.dot(w_ref[...], xtb, preferred_element_type=jnp.float32)
        pt = jnp.maximum(pt + bf_ref[...], 0.0)
        xj = x_ref[0]
        rn = lax.rsqrt(jnp.sum(xj * xj, axis=1, keepdims=True))
        cn = lax.rsqrt(jnp.sum(xtb * xtb, axis=0, keepdims=True))
        sim = rn * jnp.dot(xj, xtb, preferred_element_type=jnp.float32) * cn
        s2 = jnp.sum(pt * pt)
        mx = jnp.max(pt)
        c1 = jnp.sum(pt * sim)
        c2 = jnp.sum(sim * sim)
        lane = lax.broadcasted_iota(jnp.int32, (1, 8, 128), 2)
        sumc = jnp.where(lane == 0, s2,
                         jnp.where(lane == 2, c1,
                                   jnp.where(lane == 3, c2, 0.0)))
        mxc = jnp.where(lane == 1, mx, 0.0)
        first = jnp.logical_and(j == 0, i == 0)
        prev = jnp.where(first, 0.0, acc_ref[...])
        acc_ref[...] = jnp.where(lane == 1, jnp.maximum(prev, mxc),
                                 prev + sumc)

    return pl.pallas_call(
        body,
        grid=(B, NT, NT),
        in_specs=[
            pl.BlockSpec((1, TS, D), lambda b, j, i: (b, j, 0)),
            pl.BlockSpec((1, D, TS), lambda b, j, i: (b, 0, i)),
            pl.BlockSpec((TS, D), lambda b, j, i: (j, 0)),
            pl.BlockSpec((TS, 1), lambda b, j, i: (j, 0)),
        ],
        out_specs=pl.BlockSpec((1, 8, 128), lambda b, j, i: (b, 0, 0)),
        out_shape=jax.ShapeDtypeStruct((B, 8, 128), jnp.float32),
    )(x, xt, wfc, bfc_col)


def _adj_tile(w_ref, xt_ref, bf_ref, th_ref, eye_ref, j, i):
    pt = jnp.dot(w_ref[...], xt_ref[0], preferred_element_type=jnp.float32)
    pt = jnp.maximum(pt + bf_ref[...], 0.0)
    a = (pt >= th_ref[0, 0, 0]).astype(jnp.float32)
    sel = jnp.where(j == i, 1.0, 0.0)
    return jnp.maximum(a, eye_ref[...] * sel)


def _mask_deg(xt, wfc, bfc_col, thresh, eye):
    def body(xt_ref, w_ref, bf_ref, th_ref, eye_ref, deg_ref):
        j = pl.program_id(1)
        i = pl.program_id(2)
        a = _adj_tile(w_ref, xt_ref, bf_ref, th_ref, eye_ref, j, i)
        contrib = jnp.sum(a, axis=1, keepdims=True)

        @pl.when(i == 0)
        def _():
            deg_ref[...] = contrib

        @pl.when(i > 0)
        def _():
            deg_ref[...] += contrib

    return pl.pallas_call(
        body,
        grid=(B, NT, NT),
        in_specs=[
            pl.BlockSpec((1, D, TS), lambda b, j, i: (b, 0, i)),
            pl.BlockSpec((TS, D), lambda b, j, i: (j, 0)),
            pl.BlockSpec((TS, 1), lambda b, j, i: (j, 0)),
            pl.BlockSpec((1, 1, 1), lambda b, j, i: (b, 0, 0)),
            pl.BlockSpec((TS, TS), lambda b, j, i: (0, 0)),
        ],
        out_specs=pl.BlockSpec((TS, 1), lambda b, j, i: (b * NT + j, 0)),
        out_shape=jax.ShapeDtypeStruct((B * N, 1), jnp.float32),
    )(xt, wfc, bfc_col, thresh, eye)


def _mask_agg(xt, wfc, bfc_col, thresh, eye, y, dinv_col, bias_row, *, relu):
    def body(xt_ref, w_ref, bf_ref, th_ref, eye_ref, y_ref, d_ref, b_ref, o_ref):
        j = pl.program_id(1)
        i = pl.program_id(2)
        a = _adj_tile(w_ref, xt_ref, bf_ref, th_ref, eye_ref, j, i)
        contrib = jnp.dot(a, y_ref[0], preferred_element_type=jnp.float32)

        @pl.when(i == 0)
        def _():
            o_ref[0] = contrib

        @pl.when(i > 0)
        def _():
            o_ref[0] += contrib

        @pl.when(i == NT - 1)
        def _():
            z = d_ref[...] * o_ref[0] + b_ref[...]
            if relu:
                z = jnp.maximum(z, 0.0)
            o_ref[0] = z

    return pl.pallas_call(
        body,
        grid=(B, NT, NT),
        in_specs=[
            pl.BlockSpec((1, D, TS), lambda b, j, i: (b, 0, i)),
            pl.BlockSpec((TS, D), lambda b, j, i: (j, 0)),
            pl.BlockSpec((TS, 1), lambda b, j, i: (j, 0)),
            pl.BlockSpec((1, 1, 1), lambda b, j, i: (b, 0, 0)),
            pl.BlockSpec((TS, TS), lambda b, j, i: (0, 0)),
            pl.BlockSpec((1, TS, D), lambda b, j, i: (b, i, 0)),
            pl.BlockSpec((TS, 1), lambda b, j, i: (b * NT + j, 0)),
            pl.BlockSpec((1, D), lambda b, j, i: (0, 0)),
        ],
        out_specs=pl.BlockSpec((1, TS, D), lambda b, j, i: (b, j, 0)),
        out_shape=jax.ShapeDtypeStruct((B, N, D), jnp.float32),
    )(xt, wfc, bfc_col, thresh, eye, y, dinv_col, bias_row)


def _final_readout(agg0, y4, b4_row, wr_pad, br_pad):
    TT = 512
    NB0 = N // TT

    def body(a_ref, y_ref, b4_ref, wr_ref, br_ref, o_ref):
        i = pl.program_id(0)
        x4 = jnp.where(i < NB0, a_ref[...], y_ref[...] + b4_ref[...])
        o_ref[...] = jnp.dot(x4, wr_ref[...],
                             preferred_element_type=jnp.float32) + br_ref[...]

    return pl.pallas_call(
        body,
        grid=(B * N // TT,),
        in_specs=[
            pl.BlockSpec((TT, D), lambda i: (i % (N // TT), 0)),
            pl.BlockSpec((TT, D), lambda i: (i, 0)),
            pl.BlockSpec((1, D), lambda i: (0, 0)),
            pl.BlockSpec((D, 128), lambda i: (0, 0)),
            pl.BlockSpec((1, 128), lambda i: (0, 0)),
        ],
        out_specs=pl.BlockSpec((TT, 128), lambda i: (i, 0)),
        out_shape=jax.ShapeDtypeStruct((B * N, 128), jnp.float32),
    )(agg0, y4, b4_row, wr_pad, br_pad)


def _adjacency_scalars(acc3):
    acc = acc3[:, 0, :]
    s2, mx, c1, c2 = acc[:, 0], acc[:, 1], acc[:, 2], acc[:, 3]
    rs2 = jnp.sqrt(s2)
    gmax = jnp.max(mx / rs2)
    thresh = (0.5 * gmax) * rs2
    loss = jnp.mean(1.0 - 2.0 * (c1 / rs2) + c2)
    return thresh.reshape(B, 1, 1), loss


def kernel(inputs, std_edge, W1, b1, W2, b2, W3, b3, W4, b4, Wfc, bfc, Wr, br):
    x0 = inputs.astype(jnp.float32)
    src = std_edge[0]
    dst = std_edge[1]

    dst_w = dst.reshape(_NW, _NCH, _G)
    zrows0 = jnp.zeros((_R, D), jnp.float32)
    ones_rows = jnp.ones((_G, D), jnp.float32)
    idx_deg = jnp.stack([dst_w, dst_w], axis=-2)[None]
    degp = _sc_seg_sum(ones_rows, idx_deg, zrows0, ones_rows, gather=False)
    deg_std = degp[0, 0, :, 0] + degp[1, 0, :, 0] + 1.0
    dinv_std = (deg_std ** -0.5).reshape(N, 1)
    dinv_std_b = jnp.tile(dinv_std, (B, 1))

    bfc_col = bfc.reshape(N, 1)
    eye_ts = jnp.eye(TS, dtype=jnp.float32)
    zrows = zrows0

    y1 = _linear_rowscale(x0.reshape(B * N, D), W1.T, dinv_std_b)
    offs = (jnp.arange(B, dtype=src.dtype) * N)[:, None]
    src_wb = (src[None, :] + offs).reshape(B, _NW, _NCH, _G)
    idx1_w = jnp.stack(
        [src_wb, jnp.broadcast_to(dst_w[None], src_wb.shape)], axis=-2)
    macc1 = _sc_seg_sum(y1, idx1_w, zrows, ones_rows, gather=True)
    x1 = _edge_finalize(macc1, y1.reshape(B, N, D), dinv_std,
                        b1.reshape(1, D), relu=True)

    xt1 = x1.transpose(0, 2, 1)
    acc1 = _adj_stats(x1, xt1, Wfc, bfc_col)
    thresh1, loss1 = _adjacency_scalars(acc1)
    deg2 = _mask_deg(xt1, Wfc, bfc_col, thresh1, eye_ts)
    dinv2 = deg2 ** -0.5
    y2 = _linear_rowscale(x1.reshape(B * N, D), W2.T, dinv2)
    x2 = _mask_agg(xt1, Wfc, bfc_col, thresh1, eye_ts, y2.reshape(B, N, D),
                   dinv2, b2.reshape(1, D), relu=True)

    xt2 = x2.transpose(0, 2, 1)
    acc2 = _adj_stats(x2, xt2, Wfc, bfc_col)
    thresh2, loss2 = _adjacency_scalars(acc2)
    deg3 = _mask_deg(xt2, Wfc, bfc_col, thresh2, eye_ts)
    dinv3 = deg3 ** -0.5
    y3 = _linear_rowscale(x2.reshape(B * N, D), W3.T, dinv3)
    x3 = _mask_agg(xt2, Wfc, bfc_col, thresh2, eye_ts, y3.reshape(B, N, D),
                   dinv3, b3.reshape(1, D), relu=True)

    scale4 = jnp.concatenate(
        [dinv_std, jnp.ones(((B - 1) * N, 1), jnp.float32)], axis=0)
    y4 = _linear_rowscale(x3.reshape(B * N, D), W4.T, scale4)
    idx4_w = jnp.stack([src.reshape(_NW, _NCH, _G), dst_w], axis=-2)[None]
    macc4 = _sc_seg_sum(y4[:N], idx4_w, zrows, ones_rows, gather=True)
    agg0 = _edge_finalize(macc4, y4[:N].reshape(1, N, D), dinv_std,
                          b4.reshape(1, D), relu=False)[0]

    wr_pad = jnp.zeros((D, 128), jnp.float32).at[:, :2].set(Wr.T)
    br_pad = jnp.zeros((1, 128), jnp.float32).at[0, :2].set(br)
    ro = _final_readout(agg0, y4, b4.reshape(1, D), wr_pad, br_pad)
    out = ro[:, :2].reshape(B, N, 2)
    return out, loss1 + loss2
